# baseline (speedup 1.0000x reference)
import functools

import jax
import jax.numpy as jnp
from jax import lax
from jax.experimental import pallas as pl
from jax.experimental.pallas import tpu as pltpu
from jax.experimental.pallas import tpu_sc as plsc

B, S, H, DH, DM = 2, 512, 8, 64, 512
POS = 33 * 33
PPAD = 1152
SCALE = 0.1
EXTENT = 16.0
INV_SQRT_DH = 0.125

SBLK = 64
TBLK = 8
NSUB = 16
CHUNK = 128


def _k1_body(qt_ref, kt_ref, x_ref, gw_ref, gb_ref,
             pxc_ref, pxr_ref, pyc_ref, pyr_ref,
             pk_ref, idx_ref, vg_ref):
    q2 = qt_ref[0]
    pk_ref[0] = jnp.dot(q2, kt_ref[...],
                        preferred_element_type=jnp.float32) * INV_SQRT_DH
    xv = x_ref[0]
    vg_ref[0] = jnp.maximum(
        jnp.dot(xv, gw_ref[...], preferred_element_type=jnp.float32)
        + gb_ref[...], 0.0)
    dx = (pxc_ref[0] - pxr_ref[0]) / SCALE
    dy = (pyc_ref[0] - pyr_ref[0]) / SCALE
    gx = jnp.clip(jnp.round(dx), -EXTENT, EXTENT)
    gy = jnp.clip(jnp.round(dy), -EXTENT, EXTENT)
    idx_ref[0] = ((gx + EXTENT) + (gy + EXTENT) * 33.0).astype(jnp.int32)


def _k1(qt, kt, x, gw, gb2, pxc, pxr, pyc, pyr):
    grid = (B, S // SBLK)
    return pl.pallas_call(
        _k1_body,
        grid=grid,
        in_specs=[
            pl.BlockSpec((1, SBLK * H, DH), lambda b, i: (b, i, 0)),
            pl.BlockSpec((DH, PPAD), lambda b, i: (0, 0)),
            pl.BlockSpec((1, SBLK, DM), lambda b, i: (b, i, 0)),
            pl.BlockSpec((DM, DH), lambda b, i: (0, 0)),
            pl.BlockSpec((1, DH), lambda b, i: (0, 0)),
            pl.BlockSpec((1, SBLK, 1), lambda b, i: (b, i, 0)),
            pl.BlockSpec((1, 1, S), lambda b, i: (b, 0, 0)),
            pl.BlockSpec((1, SBLK, 1), lambda b, i: (b, i, 0)),
            pl.BlockSpec((1, 1, S), lambda b, i: (b, 0, 0)),
        ],
        out_specs=[
            pl.BlockSpec((1, SBLK * H, PPAD), lambda b, i: (b, i, 0)),
            pl.BlockSpec((1, SBLK, S), lambda b, i: (b, i, 0)),
            pl.BlockSpec((1, SBLK, DH), lambda b, i: (b, i, 0)),
        ],
        out_shape=[
            jax.ShapeDtypeStruct((B, S * H, PPAD), jnp.float32),
            jax.ShapeDtypeStruct((B, S, S), jnp.int32),
            jax.ShapeDtypeStruct((B, S, DH), jnp.float32),
        ],
    )(qt, kt, x, gw, gb2, pxc, pxr, pyc, pyr)


HP = H * PPAD
SPW = S // NSUB


def _sc_body(pk_hbm, idx_hbm, vals_hbm, ra_hbm, rv_hbm,
             idx_v, rows_v, pk_v, out_v, vals_sp,
             sem_pk0, sem_pk1, sem_g0, sem_g1,
             sem_rv0, sem_rv1, sem_ra0, sem_ra1):
    b = lax.axis_index("c")
    sid = lax.axis_index("s")
    s0 = sid * SPW
    sem_pk = (sem_pk0, sem_pk1)
    sem_g = (sem_g0, sem_g1)
    sem_rv = (sem_rv0, sem_rv1)
    sem_ra = (sem_ra0, sem_ra1)

    @pl.when(sid == 0)
    def _():
        pltpu.sync_copy(vals_hbm, vals_sp)
    plsc.subcore_barrier()

    pltpu.sync_copy(idx_hbm.at[b, pl.ds(s0 * S, SPW * S)], idx_v)

    def fire_gathers(k, p):
        for j in range(S // CHUNK):
            pltpu.async_copy(
                vals_sp.at[idx_v.at[pl.ds(k * S + j * CHUNK, CHUNK)]],
                rows_v.at[p, pl.ds(j * CHUNK, CHUNK)], sem_g[p])

    fire_gathers(0, 0)
    pltpu.async_copy(pk_hbm.at[b, s0], pk_v.at[pl.ds(0, HP)], sem_pk0)

    def pair(k2, carry):
        for p in range(2):
            k = k2 * 2 + p
            s = s0 + k
            kn = lax.rem(k + 1, SPW)
            if p == 0:
                @pl.when(k2 >= 1)
                def _():
                    pltpu.make_async_copy(
                        rows_v.at[1], rv_hbm.at[b, s - 1], sem_rv[1]).wait()
            else:
                pltpu.make_async_copy(
                    rows_v.at[0], rv_hbm.at[b, s - 1], sem_rv[0]).wait()
            fire_gathers(kn, 1 - p)
            pltpu.make_async_copy(pk_hbm.at[b, s],
                                  pk_v.at[pl.ds(p * HP, HP)],
                                  sem_pk[p]).wait()
            pltpu.async_copy(pk_hbm.at[b, s0 + kn],
                             pk_v.at[pl.ds((1 - p) * HP, HP)], sem_pk[1 - p])
            @pl.when(k2 >= 1)
            def _():
                for h in range(H):
                    pltpu.make_async_copy(
                        out_v.at[pl.ds(p * H * S + h * S, S)],
                        ra_hbm.at[b, h, s - 2], sem_ra[p]).wait()
            def chunk(c, _):
                iv = idx_v[pl.ds(k * S + c * 16, 16)]
                for h in range(H):
                    val = plsc.load_gather(pk_v, [iv + (p * HP + h * PPAD)])
                    out_v[pl.ds(p * H * S + h * S + c * 16, 16)] = val
                return _
            lax.fori_loop(0, S // 16, chunk, 0)
            for h in range(H):
                pltpu.async_copy(out_v.at[pl.ds(p * H * S + h * S, S)],
                                 ra_hbm.at[b, h, s], sem_ra[p])
            pltpu.make_async_copy(rv_hbm.at[b, s], rows_v.at[p],
                                  sem_g[p]).wait()
            pltpu.async_copy(rows_v.at[p], rv_hbm.at[b, s], sem_rv[p])
        return carry

    lax.fori_loop(0, SPW // 2, pair, 0)

    pltpu.make_async_copy(rv_hbm.at[b, s0], rows_v.at[0], sem_g0).wait()
    pltpu.make_async_copy(pk_hbm.at[b, s0],
                          pk_v.at[pl.ds(0, HP)], sem_pk0).wait()
    pltpu.make_async_copy(rows_v.at[1],
                          rv_hbm.at[b, s0 + SPW - 1], sem_rv1).wait()
    for h in range(H):
        pltpu.make_async_copy(out_v.at[pl.ds(h * S, S)],
                              ra_hbm.at[b, h, s0 + SPW - 2], sem_ra0).wait()
        pltpu.make_async_copy(out_v.at[pl.ds(H * S + h * S, S)],
                              ra_hbm.at[b, h, s0 + SPW - 1], sem_ra1).wait()


def _sc_gather(pk3, idx2, vals):
    mesh = plsc.VectorSubcoreMesh(core_axis_name="c", subcore_axis_name="s")
    f = functools.partial(
        pl.kernel, _sc_body, mesh=mesh,
        out_type=[
            jax.ShapeDtypeStruct((B, H, S, S), jnp.float32),
            jax.ShapeDtypeStruct((B, S, S, DH), jnp.float32),
        ],
        scratch_types=[
            pltpu.VMEM((SPW * S,), jnp.int32),
            pltpu.VMEM((2, S, DH), jnp.float32),
            pltpu.VMEM((2 * HP,), jnp.float32),
            pltpu.VMEM((2 * H * S,), jnp.float32),
            pltpu.VMEM_SHARED((POS, DH), jnp.float32),
            pltpu.SemaphoreType.DMA,
            pltpu.SemaphoreType.DMA,
            pltpu.SemaphoreType.DMA,
            pltpu.SemaphoreType.DMA,
            pltpu.SemaphoreType.DMA,
            pltpu.SemaphoreType.DMA,
            pltpu.SemaphoreType.DMA,
            pltpu.SemaphoreType.DMA,
        ],
        compiler_params=pltpu.CompilerParams(needs_layout_passes=False,
                                             use_tc_tiling_on_sc=False),
    )()
    return f(pk3, idx2, vals)


SP = S * DH // 128


def _k3_body(atte_ref, atto_ref, rv_ref, vg_ref, out_ref):
    g = vg_ref[0]
    for t in range(TBLK):
        blk = rv_ref[0, t] * g
        ae = atte_ref[0, :, t, :]
        ao = atto_ref[0, :, t, :]
        de = jnp.dot(ae, blk, preferred_element_type=jnp.float32)
        do = jnp.dot(ao, blk, preferred_element_type=jnp.float32)
        out_ref[0, :, t, :] = de[:, :DH] + do[:, DH:]


def _k3(atte, atto, rv128, vg128):
    grid = (B, S // TBLK)
    return pl.pallas_call(
        _k3_body,
        grid=grid,
        in_specs=[
            pl.BlockSpec((1, H, TBLK, SP), lambda b, i: (b, 0, i, 0)),
            pl.BlockSpec((1, H, TBLK, SP), lambda b, i: (b, 0, i, 0)),
            pl.BlockSpec((1, TBLK, SP, 128), lambda b, i: (b, i, 0, 0)),
            pl.BlockSpec((1, SP, 128), lambda b, i: (b, 0, 0)),
        ],
        out_specs=pl.BlockSpec((1, H, TBLK, DH), lambda b, i: (b, 0, i, 0)),
        out_shape=jax.ShapeDtypeStruct((B, H, S, DH), jnp.float32),
    )(atte, atto, rv128, vg128)


def kernel(pos, queries, att, x, keys_w, values_w, gate_w, gate_b):
    qt = queries.transpose(0, 2, 1, 3).reshape(B, S * H, DH)
    kt = jnp.pad(keys_w, ((0, PPAD - POS), (0, 0))).T
    pxc = pos[:, :, 0:1]
    pxr = pos[:, :, 0][:, None, :]
    pyc = pos[:, :, 1:2]
    pyr = pos[:, :, 1][:, None, :]
    gb2 = gate_b[None, :]

    pk, idx, vg = _k1(qt, kt, x, gate_w, gb2, pxc, pxr, pyc, pyr)
    pk3 = pk.reshape(B, S, H * PPAD)
    idx2 = idx.reshape(B, S * S)
    relattn, rv = _sc_gather(pk3, idx2, values_w)
    rv128 = rv.reshape(B, S, SP, 128)
    vg128 = vg.reshape(B, SP, 128)
    atte = att[:, :, :, 0::2]
    atto = att[:, :, :, 1::2]
    rely = _k3(atte, atto, rv128, vg128)
    return relattn, rely


# speedup vs baseline: 3.6784x; 3.6784x over previous
import functools

import jax
import jax.numpy as jnp
from jax import lax
from jax.experimental import pallas as pl
from jax.experimental.pallas import tpu as pltpu
from jax.experimental.pallas import tpu_sc as plsc

B, S, H, DH, DM = 2, 512, 8, 64, 512
POS = 33 * 33
PPAD = 1152
SCALE = 0.1
EXTENT = 16.0
INV_SQRT_DH = 0.125

SBLK = 64
TBLK = 8
NSUB = 16
CHUNK = 128


def _k1_body(qt_ref, kt_ref, x_ref, gw_ref, gb_ref,
             pxc_ref, pxr_ref, pyc_ref, pyr_ref,
             pk_ref, idx_ref, vg_ref):
    q2 = qt_ref[0]
    pk_ref[0] = jnp.dot(q2, kt_ref[...],
                        preferred_element_type=jnp.float32) * INV_SQRT_DH
    xv = x_ref[0]
    vg_ref[0] = jnp.maximum(
        jnp.dot(xv, gw_ref[...], preferred_element_type=jnp.float32)
        + gb_ref[...], 0.0)
    dx = (pxc_ref[0] - pxr_ref[0]) / SCALE
    dy = (pyc_ref[0] - pyr_ref[0]) / SCALE
    gx = jnp.clip(jnp.round(dx), -EXTENT, EXTENT)
    gy = jnp.clip(jnp.round(dy), -EXTENT, EXTENT)
    idx_ref[0] = ((gx + EXTENT) + (gy + EXTENT) * 33.0).astype(jnp.int32)


def _k1(qt, kt, x, gw, gb2, pxc, pxr, pyc, pyr):
    grid = (B, S // SBLK)
    return pl.pallas_call(
        _k1_body,
        grid=grid,
        in_specs=[
            pl.BlockSpec((1, SBLK * H, DH), lambda b, i: (b, i, 0)),
            pl.BlockSpec((DH, PPAD), lambda b, i: (0, 0)),
            pl.BlockSpec((1, SBLK, DM), lambda b, i: (b, i, 0)),
            pl.BlockSpec((DM, DH), lambda b, i: (0, 0)),
            pl.BlockSpec((1, DH), lambda b, i: (0, 0)),
            pl.BlockSpec((1, SBLK, 1), lambda b, i: (b, i, 0)),
            pl.BlockSpec((1, 1, S), lambda b, i: (b, 0, 0)),
            pl.BlockSpec((1, SBLK, 1), lambda b, i: (b, i, 0)),
            pl.BlockSpec((1, 1, S), lambda b, i: (b, 0, 0)),
        ],
        out_specs=[
            pl.BlockSpec((1, SBLK * H, PPAD), lambda b, i: (b, i, 0)),
            pl.BlockSpec((1, SBLK, S), lambda b, i: (b, i, 0)),
            pl.BlockSpec((1, SBLK, DH), lambda b, i: (b, i, 0)),
        ],
        out_shape=[
            jax.ShapeDtypeStruct((B, S * H, PPAD), jnp.float32),
            jax.ShapeDtypeStruct((B, S, S), jnp.int32),
            jax.ShapeDtypeStruct((B, S, DH), jnp.float32),
        ],
    )(qt, kt, x, gw, gb2, pxc, pxr, pyc, pyr)


HP = H * PPAD
SPW = S // NSUB


def _sc_body(pk_hbm, idx_hbm, vals_hbm, ra_hbm, rv_hbm,
             idx_v, rows_v, pk_v, out_v, vals_sp,
             sem_pk0, sem_pk1, sem_g0, sem_g1,
             sem_rv0, sem_rv1, sem_ra0, sem_ra1):
    b = lax.axis_index("c")
    sid = lax.axis_index("s")
    s0 = sid * SPW
    sem_pk = (sem_pk0, sem_pk1)
    sem_g = (sem_g0, sem_g1)
    sem_rv = (sem_rv0, sem_rv1)
    sem_ra = (sem_ra0, sem_ra1)

    @pl.when(sid == 0)
    def _():
        pltpu.sync_copy(vals_hbm, vals_sp)
    plsc.subcore_barrier()

    pltpu.sync_copy(idx_hbm.at[b, pl.ds(s0 * S, SPW * S)], idx_v)

    def fire_gathers(k, p):
        for j in range(S // CHUNK):
            pltpu.async_copy(
                vals_sp.at[idx_v.at[pl.ds(k * S + j * CHUNK, CHUNK)]],
                rows_v.at[p, pl.ds(j * CHUNK, CHUNK)], sem_g[p])

    fire_gathers(0, 0)
    pltpu.async_copy(pk_hbm.at[b, s0], pk_v.at[pl.ds(0, HP)], sem_pk0)

    def pair(k2, carry):
        for p in range(2):
            k = k2 * 2 + p
            s = s0 + k
            kn = lax.rem(k + 1, SPW)
            if p == 0:
                @pl.when(k2 >= 1)
                def _():
                    pltpu.make_async_copy(
                        vals_hbm.at[pl.ds(0, S)], rows_v.at[1],
                        sem_rv[1]).wait()
            else:
                pltpu.make_async_copy(
                    vals_hbm.at[pl.ds(0, S)], rows_v.at[0],
                    sem_rv[0]).wait()
            fire_gathers(kn, 1 - p)
            pltpu.make_async_copy(pk_hbm.at[b, s],
                                  pk_v.at[pl.ds(p * HP, HP)],
                                  sem_pk[p]).wait()
            pltpu.async_copy(pk_hbm.at[b, s0 + kn],
                             pk_v.at[pl.ds((1 - p) * HP, HP)], sem_pk[1 - p])
            @pl.when(k2 >= 1)
            def _():
                for h in range(H):
                    pltpu.make_async_copy(
                        out_v.at[pl.ds(p * H * S + h * S, S)],
                        ra_hbm.at[b, h, s - 2], sem_ra[p]).wait()
            def chunk(c, _):
                iv = idx_v[pl.ds(k * S + c * 16, 16)]
                for h in range(H):
                    val = plsc.load_gather(pk_v, [iv + (p * HP + h * PPAD)])
                    out_v[pl.ds(p * H * S + h * S + c * 16, 16)] = val
                return _
            lax.fori_loop(0, S // 16, chunk, 0)
            for h in range(H):
                pltpu.async_copy(out_v.at[pl.ds(p * H * S + h * S, S)],
                                 ra_hbm.at[b, h, s], sem_ra[p])
            pltpu.make_async_copy(vals_hbm.at[pl.ds(0, S)], rows_v.at[p],
                                  sem_g[p]).wait()
            for g in range(S // 64):
                pltpu.async_copy(rows_v.at[p, pl.ds(64 * g, 64)],
                                 rv_hbm.at[b, s, g, :, pl.ds(0, DH)],
                                 sem_rv[p])
        return carry

    lax.fori_loop(0, SPW // 2, pair, 0)

    pltpu.make_async_copy(vals_hbm.at[pl.ds(0, S)], rows_v.at[0],
                          sem_g0).wait()
    pltpu.make_async_copy(pk_hbm.at[b, s0],
                          pk_v.at[pl.ds(0, HP)], sem_pk0).wait()
    pltpu.make_async_copy(vals_hbm.at[pl.ds(0, S)], rows_v.at[1],
                          sem_rv1).wait()
    for h in range(H):
        pltpu.make_async_copy(out_v.at[pl.ds(h * S, S)],
                              ra_hbm.at[b, h, s0 + SPW - 2], sem_ra0).wait()
        pltpu.make_async_copy(out_v.at[pl.ds(H * S + h * S, S)],
                              ra_hbm.at[b, h, s0 + SPW - 1], sem_ra1).wait()


def _sc_gather(pk3, idx2, vals):
    mesh = plsc.VectorSubcoreMesh(core_axis_name="c", subcore_axis_name="s")
    f = functools.partial(
        pl.kernel, _sc_body, mesh=mesh,
        out_type=[
            jax.ShapeDtypeStruct((B, H, S, S), jnp.float32),
            jax.ShapeDtypeStruct((B, S, S // 64, 64, 128), jnp.float32),
        ],
        scratch_types=[
            pltpu.VMEM((SPW * S,), jnp.int32),
            pltpu.VMEM((2, S, DH), jnp.float32),
            pltpu.VMEM((2 * HP,), jnp.float32),
            pltpu.VMEM((2 * H * S,), jnp.float32),
            pltpu.VMEM_SHARED((POS, DH), jnp.float32),
            pltpu.SemaphoreType.DMA,
            pltpu.SemaphoreType.DMA,
            pltpu.SemaphoreType.DMA,
            pltpu.SemaphoreType.DMA,
            pltpu.SemaphoreType.DMA,
            pltpu.SemaphoreType.DMA,
            pltpu.SemaphoreType.DMA,
            pltpu.SemaphoreType.DMA,
        ],
        compiler_params=pltpu.CompilerParams(needs_layout_passes=False,
                                             use_tc_tiling_on_sc=False),
    )()
    return f(pk3, idx2, vals)


SP = S * DH // 128


def _k3_body(att_ref, rv_ref, vg_ref, out_ref):
    g = vg_ref[0]
    for t in range(TBLK):
        rows = rv_ref[0, t][:, :, 0:DH].reshape(S, DH)
        a = att_ref[0, :, t, :]
        out_ref[0, :, t, :] = jnp.dot(a, rows * g,
                                      preferred_element_type=jnp.float32)


def _k3(att, rv, vg):
    grid = (B, S // TBLK)
    return pl.pallas_call(
        _k3_body,
        grid=grid,
        in_specs=[
            pl.BlockSpec((1, H, TBLK, S), lambda b, i: (b, 0, i, 0)),
            pl.BlockSpec((1, TBLK, S // 64, 64, 128),
                         lambda b, i: (b, i, 0, 0, 0)),
            pl.BlockSpec((1, S, DH), lambda b, i: (b, 0, 0)),
        ],
        out_specs=pl.BlockSpec((1, H, TBLK, DH), lambda b, i: (b, 0, i, 0)),
        out_shape=jax.ShapeDtypeStruct((B, H, S, DH), jnp.float32),
    )(att, rv, vg)


def kernel(pos, queries, att, x, keys_w, values_w, gate_w, gate_b):
    qt = queries.transpose(0, 2, 1, 3).reshape(B, S * H, DH)
    kt = jnp.pad(keys_w, ((0, PPAD - POS), (0, 0))).T
    pxc = pos[:, :, 0:1]
    pxr = pos[:, :, 0][:, None, :]
    pyc = pos[:, :, 1:2]
    pyr = pos[:, :, 1][:, None, :]
    gb2 = gate_b[None, :]

    pk, idx, vg = _k1(qt, kt, x, gate_w, gb2, pxc, pxr, pyc, pyr)
    pk3 = pk.reshape(B, S, H * PPAD)
    idx2 = idx.reshape(B, S * S)
    relattn, rv = _sc_gather(pk3, idx2, values_w)
    rely = _k3(att, rv, vg)
    return relattn, rely


# speedup vs baseline: 3.8610x; 1.0496x over previous
import functools

import jax
import jax.numpy as jnp
from jax import lax
from jax.experimental import pallas as pl
from jax.experimental.pallas import tpu as pltpu
from jax.experimental.pallas import tpu_sc as plsc

B, S, H, DH, DM = 2, 512, 8, 64, 512
POS = 33 * 33
PPAD = 1152
SCALE = 0.1
EXTENT = 16.0
INV_SQRT_DH = 0.125

SBLK = 64
TBLK = 8
NSUB = 16
CHUNK = 128


def _k1_body(qt_ref, kt_ref, x_ref, gw_ref, gb_ref,
             pxc_ref, pxr_ref, pyc_ref, pyr_ref,
             pk_ref, idx_ref, vg_ref):
    q2 = qt_ref[0]
    pk_ref[0] = jnp.dot(q2, kt_ref[...],
                        preferred_element_type=jnp.float32) * INV_SQRT_DH
    xv = x_ref[0]
    vg_ref[0] = jnp.maximum(
        jnp.dot(xv, gw_ref[...], preferred_element_type=jnp.float32)
        + gb_ref[...], 0.0)
    dx = (pxc_ref[0] - pxr_ref[0]) / SCALE
    dy = (pyc_ref[0] - pyr_ref[0]) / SCALE
    gx = jnp.clip(jnp.round(dx), -EXTENT, EXTENT)
    gy = jnp.clip(jnp.round(dy), -EXTENT, EXTENT)
    idx_ref[0] = ((gx + EXTENT) + (gy + EXTENT) * 33.0).astype(jnp.int32)


def _k1(qt, kt, x, gw, gb2, pxc, pxr, pyc, pyr):
    grid = (B, S // SBLK)
    return pl.pallas_call(
        _k1_body,
        grid=grid,
        in_specs=[
            pl.BlockSpec((1, SBLK * H, DH), lambda b, i: (b, i, 0)),
            pl.BlockSpec((DH, PPAD), lambda b, i: (0, 0)),
            pl.BlockSpec((1, SBLK, DM), lambda b, i: (b, i, 0)),
            pl.BlockSpec((DM, DH), lambda b, i: (0, 0)),
            pl.BlockSpec((1, DH), lambda b, i: (0, 0)),
            pl.BlockSpec((1, SBLK, 1), lambda b, i: (b, i, 0)),
            pl.BlockSpec((1, 1, S), lambda b, i: (b, 0, 0)),
            pl.BlockSpec((1, SBLK, 1), lambda b, i: (b, i, 0)),
            pl.BlockSpec((1, 1, S), lambda b, i: (b, 0, 0)),
        ],
        out_specs=[
            pl.BlockSpec((1, SBLK * H, PPAD), lambda b, i: (b, i, 0)),
            pl.BlockSpec((1, SBLK, S), lambda b, i: (b, i, 0)),
            pl.BlockSpec((1, SBLK, DH), lambda b, i: (b, i, 0)),
        ],
        out_shape=[
            jax.ShapeDtypeStruct((B, S * H, PPAD), jnp.float32),
            jax.ShapeDtypeStruct((B, S, S), jnp.int32),
            jax.ShapeDtypeStruct((B, S, DH), jnp.float32),
        ],
    )(qt, kt, x, gw, gb2, pxc, pxr, pyc, pyr)


HP = H * PPAD
SPW = S // NSUB


def _sc_body(pk_hbm, idx_hbm, vals_hbm, ra_hbm, rv_hbm,
             idx_v, rows_v, pk_v, out_v, vals_sp,
             sem_pk0, sem_pk1, sem_g0, sem_g1,
             sem_rv0, sem_rv1, sem_ra0, sem_ra1):
    b = lax.axis_index("c")
    sid = lax.axis_index("s")
    s0 = sid * SPW
    sem_pk = (sem_pk0, sem_pk1)
    sem_g = (sem_g0, sem_g1)
    sem_rv = (sem_rv0, sem_rv1)
    sem_ra = (sem_ra0, sem_ra1)

    @pl.when(sid == 0)
    def _():
        pltpu.sync_copy(vals_hbm, vals_sp)
    plsc.subcore_barrier()

    pltpu.sync_copy(idx_hbm.at[b, pl.ds(s0 * S, SPW * S)], idx_v)

    def fire_gathers(k, p):
        for j in range(S // CHUNK):
            pltpu.async_copy(
                vals_sp.at[idx_v.at[pl.ds(k * S + j * CHUNK, CHUNK)]],
                rows_v.at[p, pl.ds(j * CHUNK, CHUNK)], sem_g[p])

    fire_gathers(0, 0)
    pltpu.async_copy(pk_hbm.at[b, s0], pk_v.at[pl.ds(0, HP)], sem_pk0)

    def pair(k2, carry):
        for p in range(2):
            k = k2 * 2 + p
            s = s0 + k
            kn = lax.rem(k + 1, SPW)
            if p == 0:
                @pl.when(k2 >= 1)
                def _():
                    pltpu.make_async_copy(
                        vals_hbm.at[pl.ds(0, S)], rows_v.at[1],
                        sem_rv[1]).wait()
            else:
                pltpu.make_async_copy(
                    vals_hbm.at[pl.ds(0, S)], rows_v.at[0],
                    sem_rv[0]).wait()
            fire_gathers(kn, 1 - p)
            pltpu.make_async_copy(pk_hbm.at[b, s],
                                  pk_v.at[pl.ds(p * HP, HP)],
                                  sem_pk[p]).wait()
            pltpu.async_copy(pk_hbm.at[b, s0 + kn],
                             pk_v.at[pl.ds((1 - p) * HP, HP)], sem_pk[1 - p])
            @pl.when(k2 >= 1)
            def _():
                for h in range(H):
                    pltpu.make_async_copy(
                        out_v.at[pl.ds(p * H * S + h * S, S)],
                        ra_hbm.at[b, h, s - 2], sem_ra[p]).wait()
            def chunk(c, _):
                iv = idx_v[pl.ds(k * S + c * 16, 16)]
                for h in range(H):
                    val = plsc.load_gather(pk_v, [iv + (p * HP + h * PPAD)])
                    out_v[pl.ds(p * H * S + h * S + c * 16, 16)] = val
                return _
            lax.fori_loop(0, S // 16, chunk, 0)
            for h in range(H):
                pltpu.async_copy(out_v.at[pl.ds(p * H * S + h * S, S)],
                                 ra_hbm.at[b, h, s], sem_ra[p])
            pltpu.make_async_copy(vals_hbm.at[pl.ds(0, S)], rows_v.at[p],
                                  sem_g[p]).wait()
            for g in range(S // 128):
                pltpu.async_copy(rows_v.at[p, pl.ds(64 * g, 64)],
                                 rv_hbm.at[b, s, g, :, pl.ds(0, DH)],
                                 sem_rv[p])
                pltpu.async_copy(
                    rows_v.at[p, pl.ds(S // 2 + 64 * g, 64)],
                    rv_hbm.at[b, s, g, :, pl.ds(DH, DH)], sem_rv[p])
        return carry

    lax.fori_loop(0, SPW // 2, pair, 0)

    pltpu.make_async_copy(vals_hbm.at[pl.ds(0, S)], rows_v.at[0],
                          sem_g0).wait()
    pltpu.make_async_copy(pk_hbm.at[b, s0],
                          pk_v.at[pl.ds(0, HP)], sem_pk0).wait()
    pltpu.make_async_copy(vals_hbm.at[pl.ds(0, S)], rows_v.at[1],
                          sem_rv1).wait()
    for h in range(H):
        pltpu.make_async_copy(out_v.at[pl.ds(h * S, S)],
                              ra_hbm.at[b, h, s0 + SPW - 2], sem_ra0).wait()
        pltpu.make_async_copy(out_v.at[pl.ds(H * S + h * S, S)],
                              ra_hbm.at[b, h, s0 + SPW - 1], sem_ra1).wait()


def _sc_gather(pk3, idx2, vals):
    mesh = plsc.VectorSubcoreMesh(core_axis_name="c", subcore_axis_name="s")
    f = functools.partial(
        pl.kernel, _sc_body, mesh=mesh,
        out_type=[
            jax.ShapeDtypeStruct((B, H, S, S), jnp.float32),
            jax.ShapeDtypeStruct((B, S, S // 128, 64, 128), jnp.float32),
        ],
        scratch_types=[
            pltpu.VMEM((SPW * S,), jnp.int32),
            pltpu.VMEM((2, S, DH), jnp.float32),
            pltpu.VMEM((2 * HP,), jnp.float32),
            pltpu.VMEM((2 * H * S,), jnp.float32),
            pltpu.VMEM_SHARED((POS, DH), jnp.float32),
            pltpu.SemaphoreType.DMA,
            pltpu.SemaphoreType.DMA,
            pltpu.SemaphoreType.DMA,
            pltpu.SemaphoreType.DMA,
            pltpu.SemaphoreType.DMA,
            pltpu.SemaphoreType.DMA,
            pltpu.SemaphoreType.DMA,
            pltpu.SemaphoreType.DMA,
        ],
        compiler_params=pltpu.CompilerParams(needs_layout_passes=False,
                                             use_tc_tiling_on_sc=False),
    )()
    return f(pk3, idx2, vals)


SP = S * DH // 128


def _k3_body(att_ref, rv_ref, vg_ref, out_ref):
    g = vg_ref[0]
    gl = g[0:S // 2]
    gh = g[S // 2:S]
    for t in range(TBLK):
        blk = rv_ref[0, t]
        lo = blk[:, :, 0:DH].reshape(S // 2, DH)
        hi = blk[:, :, DH:128].reshape(S // 2, DH)
        a = att_ref[0, :, t, :]
        dl = jnp.dot(a[:, 0:S // 2], lo * gl,
                     preferred_element_type=jnp.float32)
        dh = jnp.dot(a[:, S // 2:S], hi * gh,
                     preferred_element_type=jnp.float32)
        out_ref[0, :, t, :] = dl + dh


def _k3(att, rv, vg):
    grid = (B, S // TBLK)
    return pl.pallas_call(
        _k3_body,
        grid=grid,
        in_specs=[
            pl.BlockSpec((1, H, TBLK, S), lambda b, i: (b, 0, i, 0)),
            pl.BlockSpec((1, TBLK, S // 128, 64, 128),
                         lambda b, i: (b, i, 0, 0, 0)),
            pl.BlockSpec((1, S, DH), lambda b, i: (b, 0, 0)),
        ],
        out_specs=pl.BlockSpec((1, H, TBLK, DH), lambda b, i: (b, 0, i, 0)),
        out_shape=jax.ShapeDtypeStruct((B, H, S, DH), jnp.float32),
    )(att, rv, vg)


def kernel(pos, queries, att, x, keys_w, values_w, gate_w, gate_b):
    qt = queries.transpose(0, 2, 1, 3).reshape(B, S * H, DH)
    kt = jnp.pad(keys_w, ((0, PPAD - POS), (0, 0))).T
    pxc = pos[:, :, 0:1]
    pxr = pos[:, :, 0][:, None, :]
    pyc = pos[:, :, 1:2]
    pyr = pos[:, :, 1][:, None, :]
    gb2 = gate_b[None, :]

    pk, idx, vg = _k1(qt, kt, x, gate_w, gb2, pxc, pxr, pyc, pyr)
    pk3 = pk.reshape(B, S, H * PPAD)
    idx2 = idx.reshape(B, S * S)
    relattn, rv = _sc_gather(pk3, idx2, values_w)
    rely = _k3(att, rv, vg)
    return relattn, rely


# speedup vs baseline: 5.3648x; 1.3895x over previous
import functools

import jax
import jax.numpy as jnp
from jax import lax
from jax.experimental import pallas as pl
from jax.experimental.pallas import tpu as pltpu
from jax.experimental.pallas import tpu_sc as plsc

B, S, H, DH, DM = 2, 512, 8, 64, 512
POS = 33 * 33
PPAD = 1152
SCALE = 0.1
EXTENT = 16.0
INV_SQRT_DH = 0.125

SBLK = 64
TBLK = 32
NSUB = 16
CHUNK = 128


def _k1_body(qt_ref, kt_ref, x_ref, gw_ref, gb_ref,
             pxc_ref, pxr_ref, pyc_ref, pyr_ref,
             pk_ref, idx_ref, vg_ref):
    q2 = qt_ref[0]
    pk_ref[0] = (jnp.dot(q2, kt_ref[...],
                         preferred_element_type=jnp.float32)
                 * INV_SQRT_DH).reshape(SBLK * H * (PPAD // 128), 128)
    xv = x_ref[0]
    vg_ref[0] = jnp.maximum(
        jnp.dot(xv, gw_ref[...], preferred_element_type=jnp.float32)
        + gb_ref[...], 0.0)
    dx = (pxc_ref[0] - pxr_ref[0]) / SCALE
    dy = (pyc_ref[0] - pyr_ref[0]) / SCALE
    gx = jnp.clip(jnp.round(dx), -EXTENT, EXTENT)
    gy = jnp.clip(jnp.round(dy), -EXTENT, EXTENT)
    idx_ref[0] = ((gx + EXTENT) + (gy + EXTENT) * 33.0).astype(jnp.int32)


def _k1(qt, kt, x, gw, gb2, pxc, pxr, pyc, pyr):
    grid = (B, S // SBLK)
    return pl.pallas_call(
        _k1_body,
        grid=grid,
        in_specs=[
            pl.BlockSpec((1, SBLK * H, DH), lambda b, i: (b, i, 0)),
            pl.BlockSpec((DH, PPAD), lambda b, i: (0, 0)),
            pl.BlockSpec((1, SBLK, DM), lambda b, i: (b, i, 0)),
            pl.BlockSpec((DM, DH), lambda b, i: (0, 0)),
            pl.BlockSpec((1, DH), lambda b, i: (0, 0)),
            pl.BlockSpec((1, SBLK, 1), lambda b, i: (b, i, 0)),
            pl.BlockSpec((1, 1, S), lambda b, i: (b, 0, 0)),
            pl.BlockSpec((1, SBLK, 1), lambda b, i: (b, i, 0)),
            pl.BlockSpec((1, 1, S), lambda b, i: (b, 0, 0)),
        ],
        out_specs=[
            pl.BlockSpec((1, SBLK * H * (PPAD // 128), 128),
                         lambda b, i: (b, i, 0)),
            pl.BlockSpec((1, SBLK, S), lambda b, i: (b, i, 0)),
            pl.BlockSpec((1, SBLK, DH), lambda b, i: (b, i, 0)),
        ],
        out_shape=[
            jax.ShapeDtypeStruct((B, S * H * (PPAD // 128), 128),
                                 jnp.float32),
            jax.ShapeDtypeStruct((B, S, S), jnp.int32),
            jax.ShapeDtypeStruct((B, S, DH), jnp.float32),
        ],
    )(qt, kt, x, gw, gb2, pxc, pxr, pyc, pyr)


HP = H * PPAD
SPW = S // NSUB


def _sc_body(pk_hbm, idx_hbm, vals_hbm, ra_hbm, rv_hbm,
             idx_v, rows_v, pk_v, out_v, vals_sp,
             sem_pk0, sem_pk1, sem_g0, sem_g1,
             sem_rv0, sem_rv1, sem_ra0, sem_ra1):
    b = lax.axis_index("c")
    sid = lax.axis_index("s")
    s0 = sid * SPW
    sem_pk = (sem_pk0, sem_pk1)
    sem_g = (sem_g0, sem_g1)
    sem_rv = (sem_rv0, sem_rv1)
    sem_ra = (sem_ra0, sem_ra1)

    @pl.when(sid == 0)
    def _():
        pltpu.sync_copy(vals_hbm, vals_sp)
    plsc.subcore_barrier()

    pltpu.sync_copy(idx_hbm.at[b, pl.ds(s0 * S, SPW * S)], idx_v)

    def fire_gathers(k, p):
        for j in range(S // CHUNK):
            pltpu.async_copy(
                vals_sp.at[idx_v.at[pl.ds(k * S + j * CHUNK, CHUNK)]],
                rows_v.at[p, pl.ds(j * CHUNK, CHUNK)], sem_g[p])

    fire_gathers(0, 0)
    pltpu.async_copy(pk_hbm.at[b, pl.ds(s0 * (HP // 128), HP // 128)],
                     pk_v.at[0], sem_pk0)

    def pair(k2, carry):
        for p in range(2):
            k = k2 * 2 + p
            s = s0 + k
            kn = lax.rem(k + 1, SPW)
            if p == 0:
                @pl.when(k2 >= 1)
                def _():
                    pltpu.make_async_copy(
                        vals_hbm.at[pl.ds(0, S)], rows_v.at[1],
                        sem_rv[1]).wait()
            else:
                pltpu.make_async_copy(
                    vals_hbm.at[pl.ds(0, S)], rows_v.at[0],
                    sem_rv[0]).wait()
            fire_gathers(kn, 1 - p)
            pltpu.make_async_copy(
                pk_hbm.at[b, pl.ds(s * (HP // 128), HP // 128)],
                pk_v.at[p], sem_pk[p]).wait()
            pltpu.async_copy(
                pk_hbm.at[b, pl.ds((s0 + kn) * (HP // 128), HP // 128)],
                pk_v.at[1 - p], sem_pk[1 - p])
            @pl.when(k2 >= 1)
            def _():
                for h in range(H):
                    pltpu.make_async_copy(
                        out_v.at[pl.ds(p * H * S + h * S, S)],
                        ra_hbm.at[b, h, s - 2], sem_ra[p]).wait()
            def chunk(c, _):
                iv = idx_v[pl.ds(k * S + c * 16, 16)]
                vr = jax.lax.shift_right_logical(iv, 7)
                vc = jax.lax.bitwise_and(iv, 127)
                pv = jnp.full((16,), p, jnp.int32)
                for h in range(H):
                    val = plsc.load_gather(
                        pk_v, [pv, vr + (h * (PPAD // 128)), vc])
                    out_v[pl.ds(p * H * S + h * S + c * 16, 16)] = val
                return _
            lax.fori_loop(0, S // 16, chunk, 0)
            for h in range(H):
                pltpu.async_copy(out_v.at[pl.ds(p * H * S + h * S, S)],
                                 ra_hbm.at[b, h, s], sem_ra[p])
            pltpu.make_async_copy(vals_hbm.at[pl.ds(0, S)], rows_v.at[p],
                                  sem_g[p]).wait()
            for g in range(S // 128):
                pltpu.async_copy(rows_v.at[p, pl.ds(64 * g, 64)],
                                 rv_hbm.at[b, s, g, :, pl.ds(0, DH)],
                                 sem_rv[p])
                pltpu.async_copy(
                    rows_v.at[p, pl.ds(S // 2 + 64 * g, 64)],
                    rv_hbm.at[b, s, g, :, pl.ds(DH, DH)], sem_rv[p])
        return carry

    lax.fori_loop(0, SPW // 2, pair, 0)

    pltpu.make_async_copy(vals_hbm.at[pl.ds(0, S)], rows_v.at[0],
                          sem_g0).wait()
    pltpu.make_async_copy(pk_hbm.at[b, pl.ds(s0 * (HP // 128), HP // 128)],
                          pk_v.at[0], sem_pk0).wait()
    pltpu.make_async_copy(vals_hbm.at[pl.ds(0, S)], rows_v.at[1],
                          sem_rv1).wait()
    for h in range(H):
        pltpu.make_async_copy(out_v.at[pl.ds(h * S, S)],
                              ra_hbm.at[b, h, s0 + SPW - 2], sem_ra0).wait()
        pltpu.make_async_copy(out_v.at[pl.ds(H * S + h * S, S)],
                              ra_hbm.at[b, h, s0 + SPW - 1], sem_ra1).wait()


def _sc_gather(pk3, idx2, vals):
    mesh = plsc.VectorSubcoreMesh(core_axis_name="c", subcore_axis_name="s")
    f = functools.partial(
        pl.kernel, _sc_body, mesh=mesh,
        out_type=[
            jax.ShapeDtypeStruct((B, H, S, S), jnp.float32),
            jax.ShapeDtypeStruct((B, S, S // 128, 64, 128), jnp.float32),
        ],
        scratch_types=[
            pltpu.VMEM((SPW * S,), jnp.int32),
            pltpu.VMEM((2, S, DH), jnp.float32),
            pltpu.VMEM((2, HP // 128, 128), jnp.float32),
            pltpu.VMEM((2 * H * S,), jnp.float32),
            pltpu.VMEM_SHARED((POS, DH), jnp.float32),
            pltpu.SemaphoreType.DMA,
            pltpu.SemaphoreType.DMA,
            pltpu.SemaphoreType.DMA,
            pltpu.SemaphoreType.DMA,
            pltpu.SemaphoreType.DMA,
            pltpu.SemaphoreType.DMA,
            pltpu.SemaphoreType.DMA,
            pltpu.SemaphoreType.DMA,
        ],
        compiler_params=pltpu.CompilerParams(needs_layout_passes=False,
                                             use_tc_tiling_on_sc=False),
    )()
    return f(pk3, idx2, vals)


SP = S * DH // 128


def _k3_body(att_ref, rv_ref, vg_ref, out_ref):
    g = vg_ref[0]
    gl = g[0:S // 2]
    gh = g[S // 2:S]
    for t in range(TBLK):
        blk = rv_ref[0, t]
        lo = blk[:, :, 0:DH].reshape(S // 2, DH)
        hi = blk[:, :, DH:128].reshape(S // 2, DH)
        a = att_ref[0, :, t, :]
        dl = jnp.dot(a[:, 0:S // 2], lo * gl,
                     preferred_element_type=jnp.float32)
        dh = jnp.dot(a[:, S // 2:S], hi * gh,
                     preferred_element_type=jnp.float32)
        out_ref[0, :, t, :] = dl + dh


def _k3(att, rv, vg):
    grid = (B, S // TBLK)
    return pl.pallas_call(
        _k3_body,
        grid=grid,
        in_specs=[
            pl.BlockSpec((1, H, TBLK, S), lambda b, i: (b, 0, i, 0)),
            pl.BlockSpec((1, TBLK, S // 128, 64, 128),
                         lambda b, i: (b, i, 0, 0, 0)),
            pl.BlockSpec((1, S, DH), lambda b, i: (b, 0, 0)),
        ],
        out_specs=pl.BlockSpec((1, H, TBLK, DH), lambda b, i: (b, 0, i, 0)),
        out_shape=jax.ShapeDtypeStruct((B, H, S, DH), jnp.float32),
    )(att, rv, vg)


def kernel(pos, queries, att, x, keys_w, values_w, gate_w, gate_b):
    qt = queries.transpose(0, 2, 1, 3).reshape(B, S * H, DH)
    kt = jnp.pad(keys_w, ((0, PPAD - POS), (0, 0))).T
    pxc = pos[:, :, 0:1]
    pxr = pos[:, :, 0][:, None, :]
    pyc = pos[:, :, 1:2]
    pyr = pos[:, :, 1][:, None, :]
    gb2 = gate_b[None, :]

    pk, idx, vg = _k1(qt, kt, x, gate_w, gb2, pxc, pxr, pyc, pyr)
    idx2 = idx.reshape(B, S * S)
    relattn, rv = _sc_gather(pk, idx2, values_w)
    rely = _k3(att, rv, vg)
    return relattn, rely
